# baseline (speedup 1.0000x reference)
import functools

import jax
import jax.numpy as jnp
from jax.experimental import pallas as pl
from jax.experimental.pallas import tpu as pltpu

N = 4096
F_IN = 128
F_OUT = 64
NHEADS = 4
ALPHA = 0.2

BR = 256
BC = 1024


def _prep_kernel(h_ref, w_ref, a1_ref, a2_ref, wh_ref, s1_ref, s2_ref):
    wh = jnp.dot(h_ref[:], w_ref[:], preferred_element_type=jnp.float32)
    wh_ref[:] = wh
    s1_ref[:] = jnp.dot(wh, a1_ref[:], preferred_element_type=jnp.float32)
    s2_ref[:] = jnp.dot(wh, a2_ref[:], preferred_element_type=jnp.float32)


def _flash_kernel(nheads, fout, out_transform,
                  adj_ref, s1_ref, s2_ref, wh_ref, out_ref,
                  m_ref, l_ref, acc_ref):
    j = pl.program_id(1)
    nj = pl.num_programs(1)

    @pl.when(j == 0)
    def _init():
        m_ref[:] = jnp.full_like(m_ref, -jnp.inf)
        l_ref[:] = jnp.zeros_like(l_ref)
        acc_ref[:] = jnp.zeros_like(acc_ref)

    mask = adj_ref[:] > 0.0

    for k in range(nheads):
        s1 = s1_ref[:, k:k + 1]
        s2 = s2_ref[:, k:k + 1].T
        s = s1 + s2
        e = jnp.where(s >= 0.0, s, ALPHA * s)
        masked = jnp.where(mask, e, 0.0)
        bm = jnp.max(masked, axis=1, keepdims=True)
        m_old = m_ref[:, k:k + 1]
        m_new = jnp.maximum(m_old, bm)
        corr = jnp.exp(m_old - m_new)
        p = jnp.exp(masked - m_new)
        whb = wh_ref[pl.ds(j * BC, BC), k * fout:(k + 1) * fout]
        m_ref[:, k:k + 1] = m_new
        l_ref[:, k:k + 1] = l_ref[:, k:k + 1] * corr + jnp.sum(
            p, axis=1, keepdims=True)
        acc_ref[:, k * fout:(k + 1) * fout] = (
            acc_ref[:, k * fout:(k + 1) * fout] * corr
            + jnp.dot(p, whb, preferred_element_type=jnp.float32))

    @pl.when(j == nj - 1)
    def _finish():
        for k in range(nheads):
            hp = acc_ref[:, k * fout:(k + 1) * fout] / l_ref[:, k:k + 1]
            out_ref[:, k * fout:(k + 1) * fout] = out_transform(hp)


def _elu(x):
    return jnp.where(x > 0.0, x, jnp.exp(x) - 1.0)


def _elu_log_softmax(x):
    y = _elu(x)
    mx = jnp.max(y, axis=1, keepdims=True)
    lse = jnp.log(jnp.sum(jnp.exp(y - mx), axis=1, keepdims=True))
    return y - mx - lse


def _run_prep(h, wcat, a1, a2, nheads):
    n, fin = h.shape
    fcat = wcat.shape[1]
    grid = (n // BR,)
    return pl.pallas_call(
        _prep_kernel,
        grid=grid,
        in_specs=[
            pl.BlockSpec((BR, fin), lambda i: (i, 0)),
            pl.BlockSpec((fin, fcat), lambda i: (0, 0)),
            pl.BlockSpec((fcat, nheads), lambda i: (0, 0)),
            pl.BlockSpec((fcat, nheads), lambda i: (0, 0)),
        ],
        out_specs=[
            pl.BlockSpec((BR, fcat), lambda i: (i, 0)),
            pl.BlockSpec((BR, nheads), lambda i: (i, 0)),
            pl.BlockSpec((BR, nheads), lambda i: (i, 0)),
        ],
        out_shape=[
            jax.ShapeDtypeStruct((n, fcat), jnp.float32),
            jax.ShapeDtypeStruct((n, nheads), jnp.float32),
            jax.ShapeDtypeStruct((n, nheads), jnp.float32),
        ],
        compiler_params=pltpu.CompilerParams(
            dimension_semantics=("parallel",)),
    )(h, wcat, a1, a2)


def _run_flash(adj, s1, s2, wh, nheads, fout, out_transform):
    n = adj.shape[0]
    fcat = nheads * fout
    grid = (n // BR, n // BC)
    return pl.pallas_call(
        functools.partial(_flash_kernel, nheads, fout, out_transform),
        grid=grid,
        in_specs=[
            pl.BlockSpec((BR, BC), lambda i, j: (i, j)),
            pl.BlockSpec((BR, nheads), lambda i, j: (i, 0)),
            pl.BlockSpec((BC, nheads), lambda i, j: (j, 0)),
            pl.BlockSpec((n, fcat), lambda i, j: (0, 0)),
        ],
        out_specs=pl.BlockSpec((BR, fcat), lambda i, j: (i, 0)),
        out_shape=jax.ShapeDtypeStruct((n, fcat), jnp.float32),
        scratch_shapes=[
            pltpu.VMEM((BR, nheads), jnp.float32),
            pltpu.VMEM((BR, nheads), jnp.float32),
            pltpu.VMEM((BR, fcat), jnp.float32),
        ],
        compiler_params=pltpu.CompilerParams(
            dimension_semantics=("parallel", "arbitrary")),
    )(adj, s1, s2, wh)


def kernel(x, adj, W0, W1, W2, W3, a0, a1, a2, a3, Wout, aout):
    h = x.reshape(N, F_IN)
    adjm = adj.reshape(N, N)

    wcat = jnp.concatenate([W0, W1, W2, W3], axis=1)
    a_list = [a0, a1, a2, a3]
    eye = jnp.eye(NHEADS, dtype=jnp.float32)
    a1cat = jnp.concatenate(
        [a_list[k][:F_OUT] * eye[k] for k in range(NHEADS)], axis=0)
    a2cat = jnp.concatenate(
        [a_list[k][F_OUT:] * eye[k] for k in range(NHEADS)], axis=0)

    wh, s1, s2 = _run_prep(h, wcat, a1cat, a2cat, NHEADS)
    hc = _run_flash(adjm, s1, s2, wh, NHEADS, F_OUT, _elu)

    who, s1o, s2o = _run_prep(hc, Wout, aout[:F_OUT], aout[F_OUT:], 1)
    out = _run_flash(adjm, s1o, s2o, who, 1, F_OUT, _elu_log_softmax)
    return out


# speedup vs baseline: 2.2544x; 2.2544x over previous
import functools

import jax
import jax.numpy as jnp
from jax.experimental import pallas as pl
from jax.experimental.pallas import tpu as pltpu

N = 4096
F_IN = 128
F_OUT = 64
NHEADS = 4
ALPHA = 0.2

BR = 256
BC = 1024
HW = 128


def _prep_kernel(h_ref, w_ref, a1_ref, a2_ref,
                 wh_ref, s1_ref, s2t_ref, s2max_ref, runmax_ref):
    i = pl.program_id(0)
    ni = pl.num_programs(0)
    nheads = s1_ref.shape[1]
    fout = w_ref.shape[1] // nheads

    wh = jnp.dot(h_ref[:], w_ref[:], preferred_element_type=jnp.float32)
    s1 = jnp.dot(wh, a1_ref[:], preferred_element_type=jnp.float32)
    s2 = jnp.dot(wh, a2_ref[:], preferred_element_type=jnp.float32)
    s1_ref[:] = s1

    br = wh.shape[0]
    ones = jnp.ones((br, 1), jnp.float32)
    zpad = jnp.zeros((br, HW - fout - 1), jnp.float32)
    parts = []
    for k in range(nheads):
        parts += [wh[:, k * fout:(k + 1) * fout], ones, zpad]
    wh_ref[:] = jnp.concatenate(parts, axis=1)

    s2t = jnp.concatenate(
        [s2.T, jnp.zeros((8 - nheads, br), jnp.float32)], axis=0)
    s2t_ref[:] = s2t

    bmax = jnp.max(s2, axis=0, keepdims=True)

    @pl.when(i == 0)
    def _():
        runmax_ref[:] = jnp.full_like(runmax_ref, -jnp.inf)

    runmax_ref[:] = jnp.maximum(runmax_ref[:], bmax)

    @pl.when(i == ni - 1)
    def _():
        s2max_ref[:] = runmax_ref[:]


def _flash_kernel(nheads, fout, out_transform,
                  adj_ref, s1_ref, s2t_ref, s2max_ref, wh_ref,
                  out_ref, acc_ref):
    j = pl.program_id(1)
    nj = pl.num_programs(1)

    @pl.when(j == 0)
    def _init():
        acc_ref[:] = jnp.zeros_like(acc_ref)

    mask = adj_ref[:] > 0.0

    for k in range(nheads):
        s1c = s1_ref[:, k:k + 1]
        s2r = s2t_ref[k:k + 1, :]
        s2m = s2max_ref[0:1, k:k + 1]
        t = s1c + s2m
        mrow = jnp.maximum(jnp.where(t >= 0.0, t, ALPHA * t), 0.0)
        e1 = jnp.exp(t - mrow)
        f1 = jnp.exp(ALPHA * t - mrow)
        g = jnp.exp(-mrow)
        e2 = jnp.exp(s2r - s2m)
        f2 = jnp.exp(ALPHA * (s2r - s2m))

        c = s2r >= -s1c
        u = jnp.where(c, e2, f2)
        v = jnp.where(c, e1, f1)
        p = jnp.where(mask, u * v, g)
        whb = wh_ref[pl.ds(j * BC, BC), k * HW:(k + 1) * HW]
        acc_ref[:, k * HW:(k + 1) * HW] += jnp.dot(
            p, whb, preferred_element_type=jnp.float32)

    @pl.when(j == nj - 1)
    def _finish():
        for k in range(nheads):
            hp = (acc_ref[:, k * HW:k * HW + fout]
                  / acc_ref[:, k * HW + fout:k * HW + fout + 1])
            out_ref[:, k * fout:(k + 1) * fout] = out_transform(hp)


def _elu(x):
    return jnp.where(x > 0.0, x, jnp.exp(x) - 1.0)


def _elu_log_softmax(x):
    y = _elu(x)
    mx = jnp.max(y, axis=1, keepdims=True)
    lse = jnp.log(jnp.sum(jnp.exp(y - mx), axis=1, keepdims=True))
    return y - mx - lse


def _run_prep(h, wcat, a1, a2, nheads):
    n, fin = h.shape
    fcat = wcat.shape[1]
    grid = (n // BR,)
    return pl.pallas_call(
        _prep_kernel,
        grid=grid,
        in_specs=[
            pl.BlockSpec((BR, fin), lambda i: (i, 0)),
            pl.BlockSpec((fin, fcat), lambda i: (0, 0)),
            pl.BlockSpec((fcat, nheads), lambda i: (0, 0)),
            pl.BlockSpec((fcat, nheads), lambda i: (0, 0)),
        ],
        out_specs=[
            pl.BlockSpec((BR, nheads * HW), lambda i: (i, 0)),
            pl.BlockSpec((BR, nheads), lambda i: (i, 0)),
            pl.BlockSpec((8, BR), lambda i: (0, i)),
            pl.BlockSpec((1, nheads), lambda i: (0, 0)),
        ],
        out_shape=[
            jax.ShapeDtypeStruct((n, nheads * HW), jnp.float32),
            jax.ShapeDtypeStruct((n, nheads), jnp.float32),
            jax.ShapeDtypeStruct((8, n), jnp.float32),
            jax.ShapeDtypeStruct((1, nheads), jnp.float32),
        ],
        scratch_shapes=[pltpu.VMEM((1, nheads), jnp.float32)],
        compiler_params=pltpu.CompilerParams(
            dimension_semantics=("arbitrary",)),
    )(h, wcat, a1, a2)


def _run_flash(adj, s1, s2t, s2max, wh, nheads, fout, out_transform):
    n = adj.shape[0]
    grid = (n // BR, n // BC)
    return pl.pallas_call(
        functools.partial(_flash_kernel, nheads, fout, out_transform),
        grid=grid,
        in_specs=[
            pl.BlockSpec((BR, BC), lambda i, j: (i, j)),
            pl.BlockSpec((BR, nheads), lambda i, j: (i, 0)),
            pl.BlockSpec((8, BC), lambda i, j: (0, j)),
            pl.BlockSpec((1, nheads), lambda i, j: (0, 0)),
            pl.BlockSpec((n, nheads * HW), lambda i, j: (0, 0)),
        ],
        out_specs=pl.BlockSpec((BR, nheads * fout), lambda i, j: (i, 0)),
        out_shape=jax.ShapeDtypeStruct((n, nheads * fout), jnp.float32),
        scratch_shapes=[pltpu.VMEM((BR, nheads * HW), jnp.float32)],
        compiler_params=pltpu.CompilerParams(
            dimension_semantics=("parallel", "arbitrary")),
    )(adj, s1, s2t, s2max, wh)


def kernel(x, adj, W0, W1, W2, W3, a0, a1, a2, a3, Wout, aout):
    h = x.reshape(N, F_IN)
    adjm = adj.reshape(N, N)

    wcat = jnp.concatenate([W0, W1, W2, W3], axis=1)
    a_list = [a0, a1, a2, a3]
    eye = jnp.eye(NHEADS, dtype=jnp.float32)
    a1cat = jnp.concatenate(
        [a_list[k][:F_OUT] * eye[k] for k in range(NHEADS)], axis=0)
    a2cat = jnp.concatenate(
        [a_list[k][F_OUT:] * eye[k] for k in range(NHEADS)], axis=0)

    wh, s1, s2t, s2max = _run_prep(h, wcat, a1cat, a2cat, NHEADS)
    hc = _run_flash(adjm, s1, s2t, s2max, wh, NHEADS, F_OUT, _elu)

    who, s1o, s2to, s2maxo = _run_prep(hc, Wout, aout[:F_OUT], aout[F_OUT:], 1)
    out = _run_flash(adjm, s1o, s2to, s2maxo, who, 1, F_OUT, _elu_log_softmax)
    return out
